# Initial kernel scaffold; baseline (speedup 1.0000x reference)
import jax
import jax.numpy as jnp
from jax.experimental import pallas as pl


def kernel(aig_batch, xag_batch, xmg_batch, mig_batch, xag_hop_node, xmg_hop_node, mig_hop_node, xag_hop, xmg_hop, mig_hop, xag_hop_lev, xmg_hop_lev, mig_hop_lev, xag_hop_length, xmg_hop_length, mig_hop_length, tokens, masked_tokens, params):
    raise NotImplementedError("write your pallas kernel here")


# speedup vs baseline: 8.9692x; 8.9692x over previous
import functools

import jax
import jax.numpy as jnp
import numpy as np
from jax import lax
from jax.experimental import pallas as pl
from jax.experimental.pallas import tpu as pltpu
from jax.experimental.pallas import tpu_sc as plsc

D = 256
HH = 8
LMAX = 16
N = 4096
H = 512
NB = 4
HD = 32
FF = 1024
MCMF = 2048
NEG = -1e30
KEXTRA = 128
KP = N + 3 * H + KEXTRA
THOP = 128
TQ = 256
F32 = jnp.float32


def _leaky(x):
    return jnp.where(x > 0, x, 0.2 * x)


def _lnk(x, g, b):
    m = jnp.mean(x, -1, keepdims=True)
    v = jnp.mean((x - m) ** 2, -1, keepdims=True)
    return (x - m) / jnp.sqrt(v + 1e-5) * g + b


def _dot(a, b, dn=None):
    if dn is None:
        dn = (((a.ndim - 1,), (0,)), ((), ()))
    return lax.dot_general(a, b, dn, preferred_element_type=F32)


def _sc_gather(tokens_flat, midx, masked, perm):
    info = plsc.get_sparse_core_info()
    nc, ns = info.num_cores, info.num_subcores
    nw = nc * ns
    ch = 128
    m_total = midx.shape[0]
    m_per_w = m_total // nw
    p_per_w = N // nw
    mesh = plsc.VectorSubcoreMesh(core_axis_name="c", subcore_axis_name="s")

    @functools.partial(
        pl.kernel, mesh=mesh,
        out_type=[jax.ShapeDtypeStruct((m_total, D), F32),
                  jax.ShapeDtypeStruct((N, D), F32)],
        scratch_types=[pltpu.VMEM((ch,), jnp.int32),
                       pltpu.VMEM((ch, D), F32),
                       pltpu.SemaphoreType.DMA],
    )
    def k(tok_hbm, midx_hbm, masked_hbm, perm_hbm, mem_out, mp_out,
          idx_v, rows_v, sem):
        wid = lax.axis_index("s") * nc + lax.axis_index("c")
        for c in range(m_per_w // ch):
            base = wid * m_per_w + c * ch
            pltpu.sync_copy(midx_hbm.at[pl.ds(base, ch)], idx_v)
            pltpu.async_copy(tok_hbm.at[idx_v], rows_v, sem).wait()
            pltpu.sync_copy(rows_v, mem_out.at[pl.ds(base, ch)])
        for c in range(p_per_w // ch):
            base = wid * p_per_w + c * ch
            pltpu.sync_copy(perm_hbm.at[pl.ds(base, ch)], idx_v)
            pltpu.async_copy(masked_hbm.at[idx_v], rows_v, sem).wait()
            pltpu.sync_copy(rows_v, mp_out.at[pl.ds(base, ch)])

    return k(tokens_flat, midx, masked, perm)


def _hop_body(mem_ref, msk_ref, wa_ref, cdst_ref, acl_ref, hc_ref, wrh_ref,
              gbias_ref, cent_ref, n1g_ref, n1b_ref, f1w_ref, f1b_ref,
              f2w_ref, f2b_ref, n2g_ref, n2b_ref, out_ref):
    wa = wa_ref[0]
    cdst = cdst_ref[0]
    acl = acl_ref[0]
    mskf = msk_ref[0]
    amax = jnp.broadcast_to(acl, (THOP, HH))
    ams = []
    for m in range(LMAX):
        am = _leaky(_dot(mem_ref[0, :, m, :], wa) + cdst)
        ams.append(am)
        amax = jnp.maximum(amax, jnp.where(mskf[:, m:m + 1] > 0, am, NEG))
    ec = jnp.exp(acl - amax)
    s = 2.0 * ec + 1e-16
    us = [jnp.zeros((THOP, D), F32) for _ in range(HH)]
    for m in range(LMAX):
        em = jnp.where(mskf[:, m:m + 1] > 0, jnp.exp(ams[m] - amax), 0.0)
        s = s + em
        mm = mem_ref[0, :, m, :]
        for h in range(HH):
            us[h] = us[h] + em[:, h:h + 1] * mm
    gat = _dot(2.0 * ec / s, hc_ref[0])
    for h in range(HH):
        gat = gat + _dot(us[h] / s[:, h:h + 1], wrh_ref[0, h])
    gat = gat * (1.0 / HH) + gbias_ref[0]
    x = _lnk(gat + cent_ref[0], n1g_ref[0], n1b_ref[0])
    y = jnp.maximum(_dot(x, f1w_ref[0]) + f1b_ref[0], 0.0)
    y = _dot(y, f2w_ref[0]) + f2b_ref[0] + x
    out_ref[0] = _lnk(y, n2g_ref[0], n2b_ref[0])


def _hop_call(members4, mmaskf, hp):
    nt = H // THOP
    pspec = lambda shp: pl.BlockSpec((1,) + shp, lambda mk, t: (mk,) + (0,) * len(shp))
    return pl.pallas_call(
        _hop_body,
        grid=(3, nt),
        in_specs=[
            pl.BlockSpec((1, THOP, LMAX, D), lambda mk, t: (mk, t, 0, 0)),
            pl.BlockSpec((1, THOP, LMAX), lambda mk, t: (mk, t, 0)),
            pspec((D, HH)), pspec((1, HH)), pspec((1, HH)), pspec((HH, D)),
            pspec((HH, D, D)), pspec((1, D)), pspec((1, D)),
            pspec((1, D)), pspec((1, D)), pspec((D, FF)), pspec((1, FF)),
            pspec((FF, D)), pspec((1, D)), pspec((1, D)), pspec((1, D)),
        ],
        out_specs=pl.BlockSpec((1, THOP, D), lambda mk, t: (mk, t, 0)),
        out_shape=jax.ShapeDtypeStruct((3, H, D), F32),
    )(members4, mmaskf, *hp)


def _star_rows(members, maskcol, wa, cdst, acl, hc, wrh, gbias):
    am = _leaky(_dot(members, wa) + cdst)
    amax = jnp.maximum(acl, jnp.max(jnp.where(maskcol > 0, am, NEG), 0,
                                    keepdims=True))
    ec = jnp.exp(acl - amax)
    em = jnp.where(maskcol > 0, jnp.exp(am - amax), 0.0)
    s = 2.0 * ec + jnp.sum(em, 0, keepdims=True) + 1e-16
    em_s = em / s
    u = lax.dot_general(em_s, members, (((0,), (0,)), ((), ())),
                        preferred_element_type=F32)
    gat = _dot(2.0 * ec / s, hc)
    for h in range(HH):
        gat = gat + _dot(u[h:h + 1, :], wrh[h])
    return gat * (1.0 / HH) + gbias


def _tfpost(x, cent, n1g, n1b, f1w, f1b, f2w, f2b, n2g, n2b):
    x = _lnk(x + cent, n1g, n1b)
    y = jnp.maximum(_dot(x, f1w) + f1b, 0.0)
    y = _dot(y, f2w) + f2b + x
    return _lnk(y, n2g, n2b)


def _levgraph_body(ht_ref, lmT_ref, lv8_ref,
                   lwa, lcd, lacl, lhc, lwrh, lgb, lce, ln1g, ln1b, lf1w,
                   lf1b, lf2w, lf2b, ln2g, ln2b,
                   gwa, gcd, gacl, ghc, gwrh, ggb, gce, gn1g, gn1b, gf1w,
                   gf1b, gf2w, gf2b, gn2g, gn2b,
                   subg_ref, gt_ref):
    ht = ht_ref[0]
    lmT = lmT_ref[0, 0]
    rows = []
    for lev in range(4):
        g = _star_rows(ht, lmT[:, lev:lev + 1], lwa[0], lcd[0], lacl[0],
                       lhc[0], lwrh[0], lgb[0])
        rows.append(_tfpost(g, lce[0], ln1g[0], ln1b[0], lf1w[0], lf1b[0],
                            lf2w[0], lf2b[0], ln2g[0], ln2b[0]))
    rows4 = jnp.concatenate(rows, 0)
    gg = _star_rows(rows4, lv8_ref[0, 0], gwa[0], gcd[0], gacl[0], ghc[0],
                    gwrh[0], ggb[0])
    gt = _tfpost(gg, gce[0], gn1g[0], gn1b[0], gf1w[0], gf1b[0], gf2w[0],
                 gf2b[0], gn2g[0], gn2b[0])
    subg_ref[0, 0] = rows4
    gt_ref[0, 0] = gt


def _levgraph_call(ht, lmT, lv8, lp, gp):
    jspec = lambda shp: pl.BlockSpec((1,) + shp, lambda b, j: (j,) + (0,) * len(shp))
    cspec = lambda shp: pl.BlockSpec(shp, lambda b, j: (0,) * len(shp))
    stage = lambda: [jspec((D, HH)), jspec((1, HH)), jspec((1, HH)),
                     jspec((HH, D)), jspec((HH, D, D)), jspec((1, D)),
                     jspec((1, D)),
                     jspec((1, D)), jspec((1, D)), jspec((D, FF)),
                     jspec((1, FF)), jspec((FF, D)), jspec((1, D)),
                     jspec((1, D)), jspec((1, D))]
    return pl.pallas_call(
        _levgraph_body,
        grid=(NB, 3),
        in_specs=[
            pl.BlockSpec((1, H, D), lambda b, j: (j, 0, 0)),
            pl.BlockSpec((1, 1, H, 4), lambda b, j: (b, j, 0, 0)),
            pl.BlockSpec((1, 1, 4, HH), lambda b, j: (b, j, 0, 0)),
        ] + stage() + stage(),
        out_specs=[
            pl.BlockSpec((1, 1, 4, D), lambda b, j: (b, j, 0, 0)),
            pl.BlockSpec((1, 1, 1, D), lambda b, j: (b, j, 0, 0)),
        ],
        out_shape=[jax.ShapeDtypeStruct((NB, 3, 4, D), F32),
                   jax.ShapeDtypeStruct((NB, 3, 1, D), F32)],
    )(ht, lmT, lv8, *lp, *gp)


def _qkv_body(x_ref, w_ref, b_ref, o_ref):
    o_ref[...] = _dot(x_ref[...], w_ref[...]) + b_ref[...]


def _qkv_call(x, in_wT, in_b):
    tr = 384
    return pl.pallas_call(
        _qkv_body,
        grid=(KP // tr,),
        in_specs=[pl.BlockSpec((tr, D), lambda t: (t, 0)),
                  pl.BlockSpec((D, 3 * D), lambda t: (0, 0)),
                  pl.BlockSpec((1, 3 * D), lambda t: (0, 0))],
        out_specs=pl.BlockSpec((tr, 3 * D), lambda t: (t, 0)),
        out_shape=jax.ShapeDtypeStruct((KP, 3 * D), F32),
    )(x, in_wT, in_b)


def _attn_body(q_ref, kv_ref, q1h_ref, bias_ref, x0_ref,
               owT_ref, ob_ref, n1g_ref, n1b_ref, l1w_ref, l1b_ref,
               l2w_ref, l2b_ref, n2g_ref, n2b_ref, out_ref):
    rb = _dot(q1h_ref[...], bias_ref[...])
    scale = 1.0 / np.sqrt(HD)
    outs = []
    for h in range(HH):
        qh = q_ref[:, h * HD:(h + 1) * HD] * scale
        kh = kv_ref[:, h * HD:(h + 1) * HD]
        vh = kv_ref[:, D + h * HD:D + (h + 1) * HD]
        lg = lax.dot_general(qh, kh, (((1,), (1,)), ((), ())),
                             preferred_element_type=F32) + rb
        mx = jnp.max(lg, -1, keepdims=True)
        e = jnp.exp(lg - mx)
        sm = jnp.sum(e, -1, keepdims=True)
        outs.append(_dot(e, vh) / sm)
    o = jnp.concatenate(outs, -1)
    o = _dot(o, owT_ref[...]) + ob_ref[...]
    x = _lnk(x0_ref[...] + o, n1g_ref[...], n1b_ref[...])
    f = jnp.maximum(_dot(x, l1w_ref[...]) + l1b_ref[...], 0.0)
    f = _dot(f, l2w_ref[...]) + l2b_ref[...]
    out_ref[...] = _lnk(x + f, n2g_ref[...], n2b_ref[...])


def _attn_call(q, kv, q1h, bias, x0, mp):
    cspec = lambda shp: pl.BlockSpec(shp, lambda t: (0,) * len(shp))
    return pl.pallas_call(
        _attn_body,
        grid=(N // TQ,),
        in_specs=[pl.BlockSpec((TQ, D), lambda t: (t, 0)),
                  cspec((KP, 2 * D)),
                  pl.BlockSpec((TQ, NB), lambda t: (t, 0)),
                  cspec((NB, KP)),
                  pl.BlockSpec((TQ, D), lambda t: (t, 0)),
                  cspec((D, D)), cspec((1, D)), cspec((1, D)), cspec((1, D)),
                  cspec((D, MCMF)), cspec((1, MCMF)), cspec((MCMF, D)),
                  cspec((1, D)), cspec((1, D)), cspec((1, D))],
        out_specs=pl.BlockSpec((TQ, D), lambda t: (t, 0)),
        out_shape=jax.ShapeDtypeStruct((N, D), F32),
    )(q, kv, q1h, bias, x0, *mp)


def _derive(p, center):
    w = p['w']
    wr = w.reshape(D, HH, D)
    asrc = p['att_src'][0]
    adst = p['att_dst'][0]
    wa = jnp.einsum('dhk,hk->dh', wr, asrc)
    hc = (center @ w).reshape(HH, D)
    cdst = jnp.sum(hc * adst, -1)
    acl = _leaky(jnp.sum(hc * asrc, -1) + cdst)
    wrh = jnp.transpose(wr, (1, 0, 2))
    return wa, cdst[None], acl[None], hc, wrh


def _stage_params(plist, center):
    wa, cdst, acl, hc, wrh, gb, ce = [], [], [], [], [], [], []
    n1g, n1b, f1w, f1b, f2w, f2b, n2g, n2b = ([] for _ in range(8))
    for mk in range(1, 4):
        p = plist[mk]
        a, c_, l, h_, wh = _derive(p, center)
        wa.append(a); cdst.append(c_); acl.append(l); hc.append(h_)
        wrh.append(wh); gb.append(p['bias'][None]); ce.append(center[None])
        n1g.append(p['n1_g'][None]); n1b.append(p['n1_b'][None])
        f1w.append(p['ff1_w']); f1b.append(p['ff1_b'][None])
        f2w.append(p['ff2_w']); f2b.append(p['ff2_b'][None])
        n2g.append(p['n2_g'][None]); n2b.append(p['n2_b'][None])
    st = lambda xs: jnp.stack(xs, 0)
    return [st(wa), st(cdst), st(acl), st(hc), st(wrh), st(gb), st(ce),
            st(n1g), st(n1b), st(f1w), st(f1b), st(f2w), st(f2b), st(n2g),
            st(n2b)]


def kernel(aig_batch, xag_batch, xmg_batch, mig_batch, xag_hop_node,
           xmg_hop_node, mig_hop_node, xag_hop, xmg_hop, mig_hop,
           xag_hop_lev, xmg_hop_lev, mig_hop_lev, xag_hop_length,
           xmg_hop_length, mig_hop_length, tokens, masked_tokens, params):
    modal_batch = [xag_batch, xmg_batch, mig_batch]
    hop_node = [xag_hop_node, xmg_hop_node, mig_hop_node]
    hops = [xag_hop, xmg_hop, mig_hop]
    hop_lev = [xag_hop_lev, xmg_hop_lev, mig_hop_lev]
    hop_len = [xag_hop_length, xmg_hop_length, mig_hop_length]

    perm = jnp.argsort(aig_batch, stable=True)
    qb = aig_batch[perm]
    midx = jnp.concatenate(
        [(j + 1) * N + hops[j].reshape(-1) for j in range(3)], 0)
    hb = jnp.stack([modal_batch[j][hop_node[j]] for j in range(3)], 0)
    hl = jnp.stack(hop_lev, 0)
    hlen = jnp.stack(hop_len, 0)
    mmaskf = (jnp.arange(LMAX)[None, None, :] < hlen[:, :, None]).astype(F32)
    b4 = jnp.arange(NB, dtype=jnp.int32)
    lmT = ((hb[None, :, :, None] == b4[:, None, None, None])
           & (hl[None, :, :, None] == b4[None, None, None, :])).astype(F32)
    lv = lmT.max(axis=2)
    lv8 = jnp.repeat(lv[..., None], HH, -1)

    bias = jnp.full((NB, KP), NEG, F32)
    bias = bias.at[:, :N].set(jnp.where(qb[None, :] == b4[:, None], 0.0, NEG))
    bias = bias.at[:, N:N + 3 * H].set(
        jnp.where(hb.reshape(-1)[None, :] == b4[:, None], 0.0, NEG))
    owner = jnp.repeat(b4, 15)
    ev = jnp.concatenate([lv, jnp.ones((NB, 3, 1), F32)], -1).reshape(-1)
    bias = bias.at[:, N + 3 * H:N + 3 * H + 60].set(
        jnp.where((owner[None, :] == b4[:, None]) & (ev[None, :] > 0),
                  0.0, NEG))
    q1h = (qb[:, None] == b4[None, :]).astype(F32)

    members, mperm = _sc_gather(
        tokens.reshape(4 * N, D), midx.astype(jnp.int32), masked_tokens,
        perm.astype(jnp.int32))
    members4 = members.reshape(3, H, LMAX, D)

    hp = _stage_params(params['hop_tfs'], params['hop_nodes'][0])
    ht = _hop_call(members4, mmaskf, hp)

    lp = _stage_params(params['lev_tfs'], params['subg_nodes'][0])
    gp = _stage_params(params['graph_tfs'], params['graph_nodes'][0])
    subg, gt = _levgraph_call(ht, lmT, lv8, lp, gp)

    extras = jnp.concatenate([subg.reshape(NB, 3, 4, D),
                              gt.reshape(NB, 3, 1, D)], 2).reshape(60, D)
    x_keys = jnp.concatenate(
        [mperm, ht.reshape(3 * H, D), extras,
         jnp.zeros((KEXTRA - 60, D), F32)], 0)
    p = params['mcm'][0]
    qkv = _qkv_call(x_keys, p['in_w'].T, p['in_b'][None])
    q = qkv[:N, :D]
    kv = qkv[:, D:]

    mp = [p['out_w'].T, p['out_b'][None], p['n1_g'][None], p['n1_b'][None],
          p['l1_w'], p['l1_b'][None], p['l2_w'], p['l2_b'][None],
          p['n2_g'][None], p['n2_b'][None]]
    return _attn_call(q, kv, q1h, bias, mperm, mp)


# speedup vs baseline: 9.4566x; 1.0543x over previous
import functools

import jax
import jax.numpy as jnp
import numpy as np
from jax import lax
from jax.experimental import pallas as pl
from jax.experimental.pallas import tpu as pltpu
from jax.experimental.pallas import tpu_sc as plsc

D = 256
HH = 8
LMAX = 16
N = 4096
H = 512
NB = 4
HD = 32
FF = 1024
MCMF = 2048
NEG = -1e30
KEXTRA = 256
AREG = 3 * H + KEXTRA
KP = AREG + N
THOP = 128
TQ = 256
F32 = jnp.float32


def _leaky(x):
    return jnp.where(x > 0, x, 0.2 * x)


def _lnk(x, g, b):
    m = jnp.mean(x, -1, keepdims=True)
    v = jnp.mean((x - m) ** 2, -1, keepdims=True)
    return (x - m) / jnp.sqrt(v + 1e-5) * g + b


def _dot(a, b, dn=None):
    if dn is None:
        dn = (((a.ndim - 1,), (0,)), ((), ()))
    return lax.dot_general(a, b, dn, preferred_element_type=F32)


def _sc_gather_rows(table, idx):
    info = plsc.get_sparse_core_info()
    nc, ns = info.num_cores, info.num_subcores
    nw = nc * ns
    ch = 128
    total = idx.shape[0]
    per_w = total // nw
    mesh = plsc.VectorSubcoreMesh(core_axis_name="c", subcore_axis_name="s")

    @functools.partial(
        pl.kernel, mesh=mesh,
        out_type=jax.ShapeDtypeStruct((total, D), F32),
        scratch_types=[pltpu.VMEM((ch,), jnp.int32),
                       pltpu.VMEM((ch, D), F32),
                       pltpu.SemaphoreType.DMA],
    )
    def k(tab_hbm, idx_hbm, out_hbm, idx_v, rows_v, sem):
        wid = lax.axis_index("s") * nc + lax.axis_index("c")
        for c in range(per_w // ch):
            base = wid * per_w + c * ch
            pltpu.sync_copy(idx_hbm.at[pl.ds(base, ch)], idx_v)
            pltpu.async_copy(tab_hbm.at[idx_v], rows_v, sem).wait()
            pltpu.sync_copy(rows_v, out_hbm.at[pl.ds(base, ch)])

    return k(table, idx)


def _hop_body(mem_ref, msk_ref, wa_ref, cdst_ref, acl_ref, hc_ref, wrh_ref,
              gbias_ref, cent_ref, n1g_ref, n1b_ref, f1w_ref, f1b_ref,
              f2w_ref, f2b_ref, n2g_ref, n2b_ref, out_ref):
    wa = wa_ref[0]
    cdst = cdst_ref[0]
    acl = acl_ref[0]
    mskf = msk_ref[0]
    amax = jnp.broadcast_to(acl, (THOP, HH))
    ams = []
    for m in range(LMAX):
        am = _leaky(_dot(mem_ref[0, :, m, :], wa) + cdst)
        ams.append(am)
        amax = jnp.maximum(amax, jnp.where(mskf[:, m:m + 1] > 0, am, NEG))
    ec = jnp.exp(acl - amax)
    s = 2.0 * ec + 1e-16
    us = [jnp.zeros((THOP, D), F32) for _ in range(HH)]
    for m in range(LMAX):
        em = jnp.where(mskf[:, m:m + 1] > 0, jnp.exp(ams[m] - amax), 0.0)
        s = s + em
        mm = mem_ref[0, :, m, :]
        for h in range(HH):
            us[h] = us[h] + em[:, h:h + 1] * mm
    gat = _dot(2.0 * ec / s, hc_ref[0])
    for h in range(HH):
        gat = gat + _dot(us[h] / s[:, h:h + 1], wrh_ref[0, h])
    gat = gat * (1.0 / HH) + gbias_ref[0]
    x = _lnk(gat + cent_ref[0], n1g_ref[0], n1b_ref[0])
    y = jnp.maximum(_dot(x, f1w_ref[0]) + f1b_ref[0], 0.0)
    y = _dot(y, f2w_ref[0]) + f2b_ref[0] + x
    out_ref[0] = _lnk(y, n2g_ref[0], n2b_ref[0])


def _hop_call(members4, mmaskf, hp):
    nt = H // THOP
    pspec = lambda shp: pl.BlockSpec((1,) + shp, lambda mk, t: (mk,) + (0,) * len(shp))
    return pl.pallas_call(
        _hop_body,
        grid=(3, nt),
        in_specs=[
            pl.BlockSpec((1, THOP, LMAX, D), lambda mk, t: (mk, t, 0, 0)),
            pl.BlockSpec((1, THOP, LMAX), lambda mk, t: (mk, t, 0)),
            pspec((D, HH)), pspec((1, HH)), pspec((1, HH)), pspec((HH, D)),
            pspec((HH, D, D)), pspec((1, D)), pspec((1, D)),
            pspec((1, D)), pspec((1, D)), pspec((D, FF)), pspec((1, FF)),
            pspec((FF, D)), pspec((1, D)), pspec((1, D)), pspec((1, D)),
        ],
        out_specs=pl.BlockSpec((1, THOP, D), lambda mk, t: (mk, t, 0)),
        out_shape=jax.ShapeDtypeStruct((3, H, D), F32),
    )(members4, mmaskf, *hp)


def _star_rows(members, maskcol, wa, cdst, acl, hc, wrh, gbias, am=None):
    if am is None:
        am = _leaky(_dot(members, wa) + cdst)
    amax = jnp.maximum(acl, jnp.max(jnp.where(maskcol > 0, am, NEG), 0,
                                    keepdims=True))
    ec = jnp.exp(acl - amax)
    em = jnp.where(maskcol > 0, jnp.exp(am - amax), 0.0)
    s = 2.0 * ec + jnp.sum(em, 0, keepdims=True) + 1e-16
    em_s = em / s
    u = lax.dot_general(em_s, members, (((0,), (0,)), ((), ())),
                        preferred_element_type=F32)
    gat = _dot(2.0 * ec / s, hc)
    for h in range(HH):
        gat = gat + _dot(u[h:h + 1, :], wrh[h])
    return gat * (1.0 / HH) + gbias


def _tfpost(x, cent, n1g, n1b, f1w, f1b, f2w, f2b, n2g, n2b):
    x = _lnk(x + cent, n1g, n1b)
    y = jnp.maximum(_dot(x, f1w) + f1b, 0.0)
    y = _dot(y, f2w) + f2b + x
    return _lnk(y, n2g, n2b)


def _levgraph_body(ht_ref, lmT_ref, lv8_ref,
                   lwa, lcd, lacl, lhc, lwrh, lgb, lce, ln1g, ln1b, lf1w,
                   lf1b, lf2w, lf2b, ln2g, ln2b,
                   gwa, gcd, gacl, ghc, gwrh, ggb, gce, gn1g, gn1b, gf1w,
                   gf1b, gf2w, gf2b, gn2g, gn2b,
                   subg_ref, gt_ref):
    ht = ht_ref[0]
    lmT = lmT_ref[0]
    am_all = _leaky(_dot(ht, lwa[0]) + lcd[0])
    rows = []
    for c in range(NB * 4):
        rows.append(_star_rows(ht, lmT[:, c:c + 1], lwa[0], lcd[0], lacl[0],
                               lhc[0], lwrh[0], lgb[0], am=am_all))
    rows16 = _tfpost(jnp.concatenate(rows, 0), lce[0], ln1g[0], ln1b[0],
                     lf1w[0], lf1b[0], lf2w[0], lf2b[0], ln2g[0], ln2b[0])
    ggs = []
    for b in range(NB):
        ggs.append(_star_rows(rows16[b * 4:(b + 1) * 4],
                              lv8_ref[0][b * 4:(b + 1) * 4], gwa[0], gcd[0],
                              gacl[0], ghc[0], gwrh[0], ggb[0]))
    gt4 = _tfpost(jnp.concatenate(ggs, 0), gce[0], gn1g[0], gn1b[0],
                  gf1w[0], gf1b[0], gf2w[0], gf2b[0], gn2g[0], gn2b[0])
    subg_ref[0] = rows16
    gt_ref[0] = gt4


def _levgraph_call(ht, lmT2, lv8, lp, gp):
    jspec = lambda shp: pl.BlockSpec((1,) + shp, lambda j: (j,) + (0,) * len(shp))
    stage = lambda: [jspec((D, HH)), jspec((1, HH)), jspec((1, HH)),
                     jspec((HH, D)), jspec((HH, D, D)), jspec((1, D)),
                     jspec((1, D)),
                     jspec((1, D)), jspec((1, D)), jspec((D, FF)),
                     jspec((1, FF)), jspec((FF, D)), jspec((1, D)),
                     jspec((1, D)), jspec((1, D))]
    return pl.pallas_call(
        _levgraph_body,
        grid=(3,),
        in_specs=[
            pl.BlockSpec((1, H, D), lambda j: (j, 0, 0)),
            pl.BlockSpec((1, H, NB * 4), lambda j: (j, 0, 0)),
            pl.BlockSpec((1, NB * 4, HH), lambda j: (j, 0, 0)),
        ] + stage() + stage(),
        out_specs=[
            pl.BlockSpec((1, NB * 4, D), lambda j: (j, 0, 0)),
            pl.BlockSpec((1, NB, D), lambda j: (j, 0, 0)),
        ],
        out_shape=[jax.ShapeDtypeStruct((3, NB * 4, D), F32),
                   jax.ShapeDtypeStruct((3, NB, D), F32)],
    )(ht, lmT2, lv8, *lp, *gp)


def _qkv_body(x_ref, w_ref, b_ref, o_ref):
    o_ref[...] = _dot(x_ref[...], w_ref[...]) + b_ref[...]


def _qkv_call(x, in_wT, in_b):
    tr = 256
    return pl.pallas_call(
        _qkv_body,
        grid=(KP // tr,),
        in_specs=[pl.BlockSpec((tr, D), lambda t: (t, 0)),
                  pl.BlockSpec((D, 3 * D), lambda t: (0, 0)),
                  pl.BlockSpec((1, 3 * D), lambda t: (0, 0))],
        out_specs=pl.BlockSpec((tr, 3 * D), lambda t: (t, 0)),
        out_shape=jax.ShapeDtypeStruct((KP, 3 * D), F32),
    )(x, in_wT, in_b)


def _attn_body(q_ref, kv_ref, q1h_ref, bias_ref, x0_ref,
               owT_ref, ob_ref, n1g_ref, n1b_ref, l1w_ref, l1b_ref,
               l2w_ref, l2b_ref, n2g_ref, n2b_ref, out_ref):
    rb = _dot(q1h_ref[...], bias_ref[...])
    scale = 1.0 / np.sqrt(HD)
    outs = []
    for h in range(HH):
        qh = q_ref[:, h * HD:(h + 1) * HD] * scale
        kh = kv_ref[:, D + h * HD:D + (h + 1) * HD]
        vh = kv_ref[:, 2 * D + h * HD:2 * D + (h + 1) * HD]
        lg = lax.dot_general(qh, kh, (((1,), (1,)), ((), ())),
                             preferred_element_type=F32) + rb
        mx = jnp.max(lg, -1, keepdims=True)
        e = jnp.exp(lg - mx)
        sm = jnp.sum(e, -1, keepdims=True)
        outs.append(_dot(e, vh) / sm)
    o = jnp.concatenate(outs, -1)
    o = _dot(o, owT_ref[...]) + ob_ref[...]
    x = _lnk(x0_ref[...] + o, n1g_ref[...], n1b_ref[...])
    f = jnp.maximum(_dot(x, l1w_ref[...]) + l1b_ref[...], 0.0)
    f = _dot(f, l2w_ref[...]) + l2b_ref[...]
    out_ref[...] = _lnk(x + f, n2g_ref[...], n2b_ref[...])


def _attn_call(qkv, q1h, bias, x0, mp):
    cspec = lambda shp: pl.BlockSpec(shp, lambda t: (0,) * len(shp))
    return pl.pallas_call(
        _attn_body,
        grid=(N // TQ,),
        in_specs=[pl.BlockSpec((TQ, 3 * D), lambda t: (t + AREG // TQ, 0)),
                  cspec((KP, 3 * D)),
                  pl.BlockSpec((TQ, NB), lambda t: (t, 0)),
                  cspec((NB, KP)),
                  pl.BlockSpec((TQ, D), lambda t: (t, 0)),
                  cspec((D, D)), cspec((1, D)), cspec((1, D)), cspec((1, D)),
                  cspec((D, MCMF)), cspec((1, MCMF)), cspec((MCMF, D)),
                  cspec((1, D)), cspec((1, D)), cspec((1, D))],
        out_specs=pl.BlockSpec((TQ, D), lambda t: (t, 0)),
        out_shape=jax.ShapeDtypeStruct((N, D), F32),
    )(qkv, qkv, q1h, bias, x0, *mp)


def _derive(p, center):
    w = p['w']
    wr = w.reshape(D, HH, D)
    asrc = p['att_src'][0]
    adst = p['att_dst'][0]
    wa = jnp.einsum('dhk,hk->dh', wr, asrc)
    hc = (center @ w).reshape(HH, D)
    cdst = jnp.sum(hc * adst, -1)
    acl = _leaky(jnp.sum(hc * asrc, -1) + cdst)
    wrh = jnp.transpose(wr, (1, 0, 2))
    return wa, cdst[None], acl[None], hc, wrh


def _stage_params(plist, center):
    wa, cdst, acl, hc, wrh, gb, ce = [], [], [], [], [], [], []
    n1g, n1b, f1w, f1b, f2w, f2b, n2g, n2b = ([] for _ in range(8))
    for mk in range(1, 4):
        p = plist[mk]
        a, c_, l, h_, wh = _derive(p, center)
        wa.append(a); cdst.append(c_); acl.append(l); hc.append(h_)
        wrh.append(wh); gb.append(p['bias'][None]); ce.append(center[None])
        n1g.append(p['n1_g'][None]); n1b.append(p['n1_b'][None])
        f1w.append(p['ff1_w']); f1b.append(p['ff1_b'][None])
        f2w.append(p['ff2_w']); f2b.append(p['ff2_b'][None])
        n2g.append(p['n2_g'][None]); n2b.append(p['n2_b'][None])
    st = lambda xs: jnp.stack(xs, 0)
    return [st(wa), st(cdst), st(acl), st(hc), st(wrh), st(gb), st(ce),
            st(n1g), st(n1b), st(f1w), st(f1b), st(f2w), st(f2b), st(n2g),
            st(n2b)]


def kernel(aig_batch, xag_batch, xmg_batch, mig_batch, xag_hop_node,
           xmg_hop_node, mig_hop_node, xag_hop, xmg_hop, mig_hop,
           xag_hop_lev, xmg_hop_lev, mig_hop_lev, xag_hop_length,
           xmg_hop_length, mig_hop_length, tokens, masked_tokens, params):
    modal_batch = [xag_batch, xmg_batch, mig_batch]
    hop_node = [xag_hop_node, xmg_hop_node, mig_hop_node]
    hops = [xag_hop, xmg_hop, mig_hop]
    hop_lev = [xag_hop_lev, xmg_hop_lev, mig_hop_lev]
    hop_len = [xag_hop_length, xmg_hop_length, mig_hop_length]

    perm = jnp.argsort(aig_batch, stable=True)
    qb = aig_batch[perm]
    midx = jnp.concatenate(
        [(j + 1) * N + hops[j].reshape(-1) for j in range(3)], 0)
    hb = jnp.stack([modal_batch[j][hop_node[j]] for j in range(3)], 0)
    hl = jnp.stack(hop_lev, 0)
    hlen = jnp.stack(hop_len, 0)
    mmaskf = (jnp.arange(LMAX)[None, None, :] < hlen[:, :, None]).astype(F32)
    b4 = jnp.arange(NB, dtype=jnp.int32)
    lmT = ((hb[None, :, :, None] == b4[:, None, None, None])
           & (hl[None, :, :, None] == b4[None, None, None, :])).astype(F32)
    lv = lmT.max(axis=2)
    lmT2 = lmT.transpose(1, 2, 0, 3).reshape(3, H, NB * 4)
    lv8 = jnp.repeat(lv.transpose(1, 0, 2).reshape(3, NB * 4)[..., None],
                     HH, -1)

    bias = jnp.full((NB, KP), NEG, F32)
    bias = bias.at[:, :3 * H].set(
        jnp.where(hb.reshape(-1)[None, :] == b4[:, None], 0.0, NEG))
    owner = jnp.repeat(b4, 15)
    ev = jnp.concatenate([lv, jnp.ones((NB, 3, 1), F32)], -1).reshape(-1)
    bias = bias.at[:, 3 * H:3 * H + 60].set(
        jnp.where((owner[None, :] == b4[:, None]) & (ev[None, :] > 0),
                  0.0, NEG))
    bias = bias.at[:, AREG:].set(
        jnp.where(qb[None, :] == b4[:, None], 0.0, NEG))
    q1h = (qb[:, None] == b4[None, :]).astype(F32)

    members = _sc_gather_rows(tokens.reshape(4 * N, D),
                              midx.astype(jnp.int32))
    mperm = _sc_gather_rows(masked_tokens, perm.astype(jnp.int32))
    members4 = members.reshape(3, H, LMAX, D)

    hp = _stage_params(params['hop_tfs'], params['hop_nodes'][0])
    ht = _hop_call(members4, mmaskf, hp)

    lp = _stage_params(params['lev_tfs'], params['subg_nodes'][0])
    gp = _stage_params(params['graph_tfs'], params['graph_nodes'][0])
    subg, gt = _levgraph_call(ht, lmT2, lv8, lp, gp)

    extras = jnp.concatenate(
        [subg.reshape(3, NB, 4, D).transpose(1, 0, 2, 3),
         gt.transpose(1, 0, 2)[:, :, None, :]], 2).reshape(60, D)
    x_keys = jnp.concatenate(
        [ht.reshape(3 * H, D), extras, jnp.zeros((KEXTRA - 60, D), F32),
         mperm], 0)
    p = params['mcm'][0]
    qkv = _qkv_call(x_keys, p['in_w'].T, p['in_b'][None])

    mp = [p['out_w'].T, p['out_b'][None], p['n1_g'][None], p['n1_b'][None],
          p['l1_w'], p['l1_b'][None], p['l2_w'], p['l2_b'][None],
          p['n2_g'][None], p['n2_b'][None]]
    return _attn_call(qkv, q1h, bias, mperm, mp)
